# Initial kernel scaffold; baseline (speedup 1.0000x reference)
import jax
import jax.numpy as jnp
from jax.experimental import pallas as pl


def kernel(x, edge_index, W1, b1, W2, b2):
    raise NotImplementedError("write your pallas kernel here")


# speedup vs baseline: 5.3771x; 5.3771x over previous
import functools

import jax
import jax.numpy as jnp
from jax import lax
from jax.experimental import pallas as pl
from jax.experimental.pallas import tpu as pltpu
from jax.experimental.pallas import tpu_sc as plsc

NC = 2
NS = 16
NW = NC * NS
L = 16

_MESH = plsc.VectorSubcoreMesh(core_axis_name="c", subcore_axis_name="s")


@functools.lru_cache(maxsize=None)
def _deg_call(E, N):
    epw = E // NW
    nvec = epw // L
    nhz = N // L

    def body(src_hbm, dst_hbm, hs_out, hd_out, idx_v, hist):
        cid = lax.axis_index("c")
        sid = lax.axis_index("s")
        wid = cid * NS + sid
        zeros16 = jnp.zeros((L,), jnp.float32)
        ones16 = jnp.ones((L,), jnp.float32)

        def run(ind_hbm, out_hbm):
            def zero_it(i, c):
                hist[pl.ds(i * L, L)] = zeros16
                return c
            lax.fori_loop(0, nhz, zero_it, 0)
            pltpu.sync_copy(ind_hbm.at[pl.ds(wid * epw, epw)], idx_v)

            def acc(i, c):
                idx = idx_v[pl.ds(i * L, L)]
                plsc.addupdate_scatter(hist, [idx], ones16)
                return c
            lax.fori_loop(0, nvec, acc, 0)
            pltpu.sync_copy(hist, out_hbm.at[wid, 0])

        run(src_hbm, hs_out)
        run(dst_hbm, hd_out)

    return pl.kernel(
        body,
        out_type=[
            jax.ShapeDtypeStruct((NW, 1, N), jnp.float32),
            jax.ShapeDtypeStruct((NW, 1, N), jnp.float32),
        ],
        mesh=_MESH,
        scratch_types=[
            pltpu.VMEM((epw,), jnp.int32),
            pltpu.VMEM((N,), jnp.float32),
        ],
        compiler_params=pltpu.CompilerParams(needs_layout_passes=False),
    )


@functools.lru_cache(maxsize=None)
def _agg_call(E, N, D, CH=80):
    epw = E // NW
    nch = epw // CH
    Npad = -(-N // 128) * 128
    rpt = Npad // NS

    def body(h_hbm, src_hbm, dst_hbm, zz_hbm, out_hbm, sidx, didx, rows, agg_sh, sem):
        cid = lax.axis_index("c")
        sid = lax.axis_index("s")
        wid = cid * NS + sid
        pltpu.sync_copy(zz_hbm.at[pl.ds(sid * rpt, rpt)],
                        agg_sh.at[pl.ds(sid * rpt, rpt)])
        plsc.subcore_barrier()

        def step(i, c):
            base = wid * epw + i * CH
            pltpu.sync_copy(src_hbm.at[pl.ds(base, CH)], sidx)
            pltpu.sync_copy(dst_hbm.at[pl.ds(base, CH)], didx)
            pltpu.async_copy(h_hbm.at[sidx], rows, sem).wait()
            pltpu.sync_copy(rows, agg_sh.at[didx], add=True)
            return c
        lax.fori_loop(0, nch, step, 0)

        plsc.subcore_barrier()
        pltpu.sync_copy(agg_sh.at[pl.ds(sid * rpt, rpt)],
                        out_hbm.at[cid, pl.ds(sid * rpt, rpt)])

    return pl.kernel(
        body,
        out_type=jax.ShapeDtypeStruct((NC, Npad, D), jnp.float32),
        mesh=_MESH,
        scratch_types=[
            pltpu.VMEM((CH,), jnp.int32),
            pltpu.VMEM((CH,), jnp.int32),
            pltpu.VMEM((CH, D), jnp.float32),
            pltpu.VMEM_SHARED((Npad, D), jnp.float32),
            pltpu.SemaphoreType.DMA,
        ],
        compiler_params=pltpu.CompilerParams(needs_layout_passes=False),
    )


def _prep_body(x_ref, hs_ref, hd_ref, h1_ref, ns_ref, nd_ref):
    ds = jnp.sum(hs_ref[...], axis=1, keepdims=True)
    dd = jnp.sum(hd_ref[...], axis=1, keepdims=True)
    ns = jnp.where(ds > 0, lax.rsqrt(ds), 0.0)
    nd = jnp.where(dd > 0, lax.rsqrt(dd), 0.0)
    ns_ref[...] = ns
    nd_ref[...] = nd
    h1_ref[...] = x_ref[...] * ns


@functools.lru_cache(maxsize=None)
def _prep_call(N, D, R=400):
    grid = N // R
    return pl.pallas_call(
        _prep_body,
        grid=(grid,),
        in_specs=[
            pl.BlockSpec((R, D), lambda i: (i, 0)),
            pl.BlockSpec((R, NW), lambda i: (i, 0)),
            pl.BlockSpec((R, NW), lambda i: (i, 0)),
        ],
        out_specs=[
            pl.BlockSpec((R, D), lambda i: (i, 0)),
            pl.BlockSpec((R, 1), lambda i: (i, 0)),
            pl.BlockSpec((R, 1), lambda i: (i, 0)),
        ],
        out_shape=[
            jax.ShapeDtypeStruct((N, D), jnp.float32),
            jax.ShapeDtypeStruct((N, 1), jnp.float32),
            jax.ShapeDtypeStruct((N, 1), jnp.float32),
        ],
    )


def _dense_body(apply_tanh, agg_ref, nd_ref, ns_ref, w_ref, b_ref, out_ref):
    a = (agg_ref[0] + agg_ref[1]) * nd_ref[...]
    y = jnp.dot(a, w_ref[...], preferred_element_type=jnp.float32,
                precision=lax.Precision.HIGHEST) + b_ref[...]
    if apply_tanh:
        y = jnp.tanh(y) * ns_ref[...]
    out_ref[...] = y


@functools.lru_cache(maxsize=None)
def _dense_call(N, D, H, apply_tanh, R=400):
    grid = N // R
    return pl.pallas_call(
        functools.partial(_dense_body, apply_tanh),
        grid=(grid,),
        in_specs=[
            pl.BlockSpec((NC, R, D), lambda i: (0, i, 0)),
            pl.BlockSpec((R, 1), lambda i: (i, 0)),
            pl.BlockSpec((R, 1), lambda i: (i, 0)),
            pl.BlockSpec((D, H), lambda i: (0, 0)),
            pl.BlockSpec((1, H), lambda i: (0, 0)),
        ],
        out_specs=pl.BlockSpec((R, H), lambda i: (i, 0)),
        out_shape=jax.ShapeDtypeStruct((N, H), jnp.float32),
    )


def kernel(x, edge_index, W1, b1, W2, b2):
    N, D = x.shape
    H = W1.shape[1]
    E = edge_index.shape[1]
    src = edge_index[0]
    dst = edge_index[1]

    hs, hd = _deg_call(E, N)(src, dst)
    h1, ns, nd = _prep_call(N, D)(x, hs[:, 0, :].T, hd[:, 0, :].T)

    Npad = -(-N // 128) * 128
    zz = jnp.zeros((Npad, D), jnp.float32)
    agg1 = _agg_call(E, N, D)(h1, src, dst, zz)
    h2 = _dense_call(N, D, H, True)(agg1, nd, ns, W1, b1[None])

    agg2 = _agg_call(E, N, H)(h2, src, dst, zz)
    half = D // 2
    W2e = jnp.concatenate([W2[:, half:], -W2[:, :half]], axis=1)
    b2e = jnp.concatenate([b2[half:], -b2[:half]])
    out = _dense_call(N, H, D, False)(agg2, nd, ns, W2e, b2e[None])
    return out
